# baseline (speedup 1.0000x reference)
import functools

import jax
import jax.numpy as jnp
from jax import lax
from jax.experimental import pallas as pl
from jax.experimental.pallas import tpu as pltpu
from jax.experimental.pallas import tpu_sc as plsc

DIM = 768
EO, EI = 8, 8
HIDDEN = 768
N = 2048
EPS = 1e-9
CAP_O = 320
CAP_I = 50
CAP_I_PAD = 64
ROWS_E = EI * CAP_I_PAD
ROWS = EO * ROWS_E
SENT = N
_PH = lax.Precision.HIGHEST


def _split3(v):
    h = v.astype(jnp.bfloat16)
    r1 = v - h.astype(jnp.float32)
    m = r1.astype(jnp.bfloat16)
    r2 = r1 - m.astype(jnp.float32)
    return h, m, r2.astype(jnp.bfloat16)


def _dot_parts(a, parts, dims, split_rhs=True):
    out = None
    for part in parts:
        term = lax.dot_general(
            a if split_rhs else part, part if split_rhs else a,
            dims, preferred_element_type=jnp.float32)
        out = term if out is None else out + term
    return out


def _dot_split(a, b, dims, split_rhs):
    parts = _split3(b) if split_rhs else _split3(a)
    return _dot_parts(a if split_rhs else b, parts, dims, split_rhs)


def _first_argmax(vals, lanes, num):
    m = jnp.max(vals, axis=-1, keepdims=True)
    idx = jnp.min(jnp.where(vals == m, lanes, num), axis=-1, keepdims=True)
    return m, idx


def _tri(n):
    r = lax.broadcasted_iota(jnp.int32, (n, n), 0)
    c = lax.broadcasted_iota(jnp.int32, (n, n), 1)
    return (r >= c).astype(jnp.bfloat16)


def _cumsum0(tri, mask):
    return jnp.dot(tri, mask.astype(jnp.bfloat16),
                   preferred_element_type=jnp.float32)


def _cumsum0_chunked(tri_ch, mask, ch):
    n, k = mask.shape
    off = jnp.zeros((1, k), jnp.float32)
    outs = []
    for c in range(n // ch):
        cs = _cumsum0(tri_ch, mask[c * ch:(c + 1) * ch]) + off
        outs.append(cs)
        off = cs[ch - 1:ch, :]
    return jnp.concatenate(outs, axis=0)


def _outer_body(x_ref, wgo_ref, wgi_ref, td_ref, sm_ref, losso_ref):
    x = x_ref[...]
    lane8 = lax.broadcasted_iota(jnp.int32, (N, EO), 1)

    logits = jnp.dot(x, wgo_ref[...], preferred_element_type=jnp.float32)
    mx = jnp.max(logits, axis=-1, keepdims=True)
    ex = jnp.exp(logits - mx)
    raw = ex / jnp.sum(ex, axis=-1, keepdims=True)

    g1, i1 = _first_argmax(raw, lane8, EO)
    mask1 = (lane8 == i1).astype(jnp.float32)
    raw_wo = raw * (1.0 - mask1)
    g2, i2 = _first_argmax(raw_wo, lane8, EO)
    mask2 = (lane8 == i2).astype(jnp.float32)
    denom = g1 + g2 + EPS
    g1n = g1 / denom
    g2n = g2 / denom

    loss_o = jnp.sum(jnp.mean(mask1, axis=0) * jnp.mean(raw, axis=0)) * 8.0

    tri256 = _tri(256)
    cs1 = _cumsum0_chunked(tri256, mask1, 256)
    pos1 = (cs1 - mask1) * mask1
    mask1c = mask1 * (pos1 < float(CAP_O)).astype(jnp.float32)
    count1 = jnp.sum(mask1c, axis=0, keepdims=True)
    m1flat = jnp.sum(mask1c, axis=-1, keepdims=True)
    p1f = jnp.sum(pos1, axis=-1, keepdims=True)
    g1f = g1n * m1flat

    cs2 = _cumsum0_chunked(tri256, mask2, 256)
    pos2 = ((cs2 - mask2) + count1) * mask2
    mask2c = mask2 * (pos2 < float(CAP_O)).astype(jnp.float32)
    m2flat = jnp.sum(mask2c, axis=-1, keepdims=True)
    p2f = jnp.sum(pos2, axis=-1, keepdims=True)
    g2f = g2n * m2flat

    wgi = wgi_ref[...]
    sm_parts = []
    for e in range(EO):
        li = jnp.dot(x, wgi[e], preferred_element_type=jnp.float32)
        mi = jnp.max(li, axis=-1, keepdims=True)
        ei_ = jnp.exp(li - mi)
        sm_parts.append(ei_ / jnp.sum(ei_, axis=-1, keepdims=True))
    sm_ref[...] = jnp.concatenate(sm_parts, axis=1)

    td_ref[...] = jnp.concatenate(
        [i1.astype(jnp.float32), p1f, m1flat, g1f,
         i2.astype(jnp.float32), p2f, m2flat, g2f], axis=1)
    losso_ref[...] = jnp.reshape(loss_o, (1, 1))


def _inner_body(td_ref, sm_ref, src_ref, tok1_ref, tok2_ref, ci_ref, cw_ref,
                lossi_ref):
    e = pl.program_id(0)
    td = td_ref[...]
    i1 = td[:, 0:1].astype(jnp.int32)
    p1 = td[:, 1:2].astype(jnp.int32)
    m1flat = td[:, 2:3]
    g1f = td[:, 3:4]
    i2 = td[:, 4:5].astype(jnp.int32)
    p2 = td[:, 5:6].astype(jnp.int32)
    m2flat = td[:, 6:7]
    g2f = td[:, 7:8]

    cap_io = lax.broadcasted_iota(jnp.int32, (N, CAP_O), 1)
    lane8i = lax.broadcasted_iota(jnp.int32, (CAP_O, EI), 1)
    rows_io = lax.broadcasted_iota(jnp.int32, (CAP_O, ROWS_E), 1)
    n_iota = lax.broadcasted_iota(jnp.int32, (1, N), 1).astype(jnp.float32)
    sel_r = lax.broadcasted_iota(jnp.int32, (EO * EI, EI), 0)
    sel_c = lax.broadcasted_iota(jnp.int32, (EO * EI, EI), 1)
    tri320 = _tri(CAP_O)
    smp = _split3(sm_ref[...])
    np3 = _split3(n_iota)
    g1fp = _split3(g1f)
    g2fp = _split3(g2f)

    sel1 = (i1 == e) & (m1flat > 0.0)
    oh1 = ((cap_io == p1) & sel1).astype(jnp.bfloat16)
    sel2 = (i2 == e) & (m2flat > 0.0)
    oh2 = ((cap_io == p2) & sel2).astype(jnp.bfloat16)
    oh = oh1 + oh2

    cnt = jnp.dot(jnp.ones((1, N), jnp.bfloat16), oh,
                  preferred_element_type=jnp.float32)
    src_o = (_dot_parts(oh, np3, (((1,), (0,)), ((), ())), False)
             + (1.0 - cnt) * float(SENT))

    sel_mat = (sel_r == sel_c + 8 * e).astype(jnp.float32)
    G_all = _dot_parts(oh, smp, (((0,), (0,)), ((), ())))
    G = jnp.dot(G_all, sel_mat, preferred_element_type=jnp.float32,
                precision=_PH)
    imp = (_dot_parts(oh1, g1fp, (((0,), (0,)), ((), ())))
           + _dot_parts(oh2, g2fp, (((0,), (0,)), ((), ()))))
    eq1 = (imp > 0.5).astype(jnp.float32)
    gt0 = (imp > 0.0).astype(jnp.float32)

    ig1, ii1 = _first_argmax(G, lane8i, EI)
    im1 = (lane8i == ii1).astype(jnp.float32) * eq1
    ig1 = ig1 * eq1
    G_wo = G * (1.0 - im1)
    ig2, ii2 = _first_argmax(G_wo, lane8i, EI)
    im2 = (lane8i == ii2).astype(jnp.float32) * gt0
    idenom = ig1 + ig2 + EPS
    ig1n = ig1 / idenom
    ig2n = ig2 / idenom

    loss_e = jnp.sum(jnp.mean(im1, axis=0) * jnp.mean(G * eq1, axis=0))

    csi1 = _cumsum0(tri320, im1)
    ipos1 = (csi1 - im1) * im1
    im1c = im1 * (ipos1 < float(CAP_I)).astype(jnp.float32)
    icount1 = jnp.sum(im1c, axis=0, keepdims=True)
    im1flat = jnp.sum(im1c, axis=-1, keepdims=True)
    q1 = jnp.sum(ipos1, axis=-1, keepdims=True).astype(jnp.int32)
    h1 = ig1n * im1flat

    csi2 = _cumsum0(tri320, im2)
    ipos2 = ((csi2 - im2) + icount1) * im2
    im2c = im2 * (ipos2 < float(CAP_I)).astype(jnp.float32)
    im2flat = jnp.sum(im2c, axis=-1, keepdims=True)
    q2 = jnp.sum(ipos2, axis=-1, keepdims=True).astype(jnp.int32)
    h2 = ig2n * im2flat

    r1 = ii1 * CAP_I_PAD + q1
    r2 = ii2 * CAP_I_PAD + q2
    ohi = (((rows_io == r1) & (im1flat > 0.0)).astype(jnp.bfloat16)
           + ((rows_io == r2) & (im2flat > 0.0)).astype(jnp.bfloat16))
    cnti = jnp.dot(jnp.ones((1, CAP_O), jnp.bfloat16), ohi,
                   preferred_element_type=jnp.float32)
    src_e = (_dot_split(src_o, ohi, (((1,), (0,)), ((), ())), False)
             + (1.0 - cnti) * float(SENT))
    src_ref[...] = jnp.reshape(src_e.astype(jnp.int32), (1, 1, ROWS_E))

    base = (e * ROWS_E).astype(jnp.float32)
    V = jnp.concatenate(
        [h1, h2, r1.astype(jnp.float32) + base,
         r2.astype(jnp.float32) + base], axis=1)
    d1 = _dot_split(oh1, V, (((1,), (0,)), ((), ())), True)
    d2 = _dot_split(oh2, V, (((1,), (0,)), ((), ())), True)

    @pl.when(e == 0)
    def _():
        tok1_ref[...] = jnp.zeros((N, 4), jnp.float32)
        tok2_ref[...] = jnp.zeros((N, 4), jnp.float32)
        lossi_ref[...] = jnp.zeros((1, 1), jnp.float32)

    tok1_ref[...] += d1
    tok2_ref[...] += d2
    lossi_ref[...] += jnp.reshape(loss_e, (1, 1))

    @pl.when(e == EO - 1)
    def _():
        tok1 = tok1_ref[...]
        tok2 = tok2_ref[...]
        cw = jnp.concatenate(
            [g1f * tok1[:, 0:1], g1f * tok1[:, 1:2],
             g2f * tok2[:, 0:1], g2f * tok2[:, 1:2]], axis=1)
        ci = jnp.concatenate(
            [tok1[:, 2:3], tok1[:, 3:4], tok2[:, 2:3], tok2[:, 3:4]],
            axis=1).astype(jnp.int32)
        ci_ref[...] = jnp.clip(jnp.where(cw != 0.0, ci, 0), 0, ROWS - 1)
        cw_ref[...] = cw


def _ffn_body(x_ref, w1_ref, w2_ref, y_ref):
    h = jnp.maximum(
        jnp.dot(x_ref[...], w1_ref[0], preferred_element_type=jnp.float32),
        0.0)
    y_ref[...] = jnp.dot(h, w2_ref[0], preferred_element_type=jnp.float32)


def _combine_body(yg_ref, w_ref, out_ref):
    acc = w_ref[0][:, None] * yg_ref[0]
    for j in range(1, 4):
        acc = acc + w_ref[j][:, None] * yg_ref[j]
    out_ref[...] = acc


def _sc_gather(table, idx):
    b = idx.shape[0]
    d = table.shape[1]
    nw = 32
    b_per_w = b // nw
    ch = 64
    nch = b_per_w // ch
    idx2d = idx.reshape(b // ch, ch)
    mesh = plsc.VectorSubcoreMesh(core_axis_name="c", subcore_axis_name="s")

    @functools.partial(
        pl.kernel, mesh=mesh,
        out_type=jax.ShapeDtypeStruct((b, d), jnp.float32),
        scratch_types=[
            pltpu.VMEM((nch, ch), jnp.int32),
            pltpu.VMEM((ch, d), jnp.float32),
            pltpu.VMEM((ch, d), jnp.float32),
            pltpu.SemaphoreType.DMA,
            pltpu.SemaphoreType.DMA,
            pltpu.SemaphoreType.DMA,
            pltpu.SemaphoreType.DMA,
        ])
    def k(table_hbm, idx_hbm, out_hbm, idx_v, rows0, rows1, g0, g1, s0, s1):
        wid = lax.axis_index("s") * 2 + lax.axis_index("c")
        rows = (rows0, rows1)
        gsem = (g0, g1)
        ssem = (s0, s1)
        pltpu.sync_copy(idx_hbm.at[pl.ds(wid * nch, nch)], idx_v)

        def gather(c):
            return pltpu.async_copy(
                table_hbm.at[idx_v.at[c]], rows[c % 2], gsem[c % 2])

        def store(c):
            return pltpu.async_copy(
                rows[c % 2],
                out_hbm.at[pl.ds(wid * b_per_w + c * ch, ch)],
                ssem[c % 2])

        gathers = [None] * nch
        stores = [None] * nch
        gathers[0] = gather(0)
        for c in range(nch):
            gathers[c].wait()
            if c + 1 < nch:
                if c >= 1:
                    stores[c - 1].wait()
                gathers[c + 1] = gather(c + 1)
            stores[c] = store(c)
        for c in range(max(0, nch - 2), nch):
            stores[c].wait()

    return k(table, idx2d)


_outer_call = pl.pallas_call(
    _outer_body,
    out_shape=(
        jax.ShapeDtypeStruct((N, 8), jnp.float32),
        jax.ShapeDtypeStruct((N, EO * EI), jnp.float32),
        jax.ShapeDtypeStruct((1, 1), jnp.float32),
    ))

_inner_call = pl.pallas_call(
    _inner_body,
    grid=(EO,),
    in_specs=[
        pl.BlockSpec((N, 8), lambda e: (0, 0)),
        pl.BlockSpec((N, EO * EI), lambda e: (0, 0)),
    ],
    out_specs=(
        pl.BlockSpec((1, 1, ROWS_E), lambda e: (e, 0, 0)),
        pl.BlockSpec((N, 4), lambda e: (0, 0)),
        pl.BlockSpec((N, 4), lambda e: (0, 0)),
        pl.BlockSpec((N, 4), lambda e: (0, 0)),
        pl.BlockSpec((N, 4), lambda e: (0, 0)),
        pl.BlockSpec((1, 1), lambda e: (0, 0)),
    ),
    out_shape=(
        jax.ShapeDtypeStruct((EO, 1, ROWS_E), jnp.int32),
        jax.ShapeDtypeStruct((N, 4), jnp.float32),
        jax.ShapeDtypeStruct((N, 4), jnp.float32),
        jax.ShapeDtypeStruct((N, 4), jnp.int32),
        jax.ShapeDtypeStruct((N, 4), jnp.float32),
        jax.ShapeDtypeStruct((1, 1), jnp.float32),
    ))

_ffn_call = pl.pallas_call(
    _ffn_body,
    grid=(EO * EI,),
    in_specs=[
        pl.BlockSpec((CAP_I_PAD, DIM), lambda i: (i, 0)),
        pl.BlockSpec((1, DIM, HIDDEN), lambda i: (i, 0, 0)),
        pl.BlockSpec((1, HIDDEN, DIM), lambda i: (i, 0, 0)),
    ],
    out_specs=pl.BlockSpec((CAP_I_PAD, DIM), lambda i: (i, 0)),
    out_shape=jax.ShapeDtypeStruct((ROWS, DIM), jnp.float32))

_combine_call = pl.pallas_call(
    _combine_body,
    grid=(8,),
    in_specs=[
        pl.BlockSpec((4, N // 8, DIM), lambda i: (0, i, 0)),
        pl.BlockSpec((4, N // 8), lambda i: (0, i)),
    ],
    out_specs=pl.BlockSpec((N // 8, DIM), lambda i: (i, 0)),
    out_shape=jax.ShapeDtypeStruct((N, DIM), jnp.float32))


def kernel(inputs, wg_outer, wg_inner, w1, w2):
    x2d = inputs.reshape(N, DIM)
    td, sm_all, loss_o = _outer_call(x2d, wg_outer, wg_inner)
    src, _t1, _t2, ci, cw, loss_i = _inner_call(td, sm_all)
    loss = (loss_o + loss_i) * 0.01

    xpad = jnp.concatenate([x2d, jnp.zeros((8, DIM), jnp.float32)], axis=0)
    xg = _sc_gather(xpad, src.reshape(ROWS))

    y = _ffn_call(xg, w1.reshape(EO * EI, DIM, HIDDEN),
                  w2.reshape(EO * EI, HIDDEN, DIM))

    yg = _sc_gather(y, ci.T.reshape(4 * N))
    out = _combine_call(yg.reshape(4, N, DIM), cw.T)
    return out.reshape(1, N, DIM), loss[0, 0]


# speedup vs baseline: 1.5729x; 1.5729x over previous
import functools

import jax
import jax.numpy as jnp
from jax import lax
from jax.experimental import pallas as pl
from jax.experimental.pallas import tpu as pltpu
from jax.experimental.pallas import tpu_sc as plsc

DIM = 768
EO, EI = 8, 8
HIDDEN = 768
N = 2048
EPS = 1e-9
CAP_O = 320
CAP_I = 50
CAP_I_PAD = 64
ROWS_E = EI * CAP_I_PAD
ROWS = EO * ROWS_E
SENT = N
_PH = lax.Precision.HIGHEST


def _split3(v):
    h = v.astype(jnp.bfloat16)
    r1 = v - h.astype(jnp.float32)
    m = r1.astype(jnp.bfloat16)
    r2 = r1 - m.astype(jnp.float32)
    return h, m, r2.astype(jnp.bfloat16)


def _dot_parts(a, parts, dims, split_rhs=True):
    out = None
    for part in parts:
        term = lax.dot_general(
            a if split_rhs else part, part if split_rhs else a,
            dims, preferred_element_type=jnp.float32)
        out = term if out is None else out + term
    return out


def _dot_split(a, b, dims, split_rhs):
    parts = _split3(b) if split_rhs else _split3(a)
    return _dot_parts(a if split_rhs else b, parts, dims, split_rhs)


def _first_argmax(vals, lanes, num):
    m = jnp.max(vals, axis=-1, keepdims=True)
    idx = jnp.min(jnp.where(vals == m, lanes, num), axis=-1, keepdims=True)
    return m, idx


def _tri(n):
    r = lax.broadcasted_iota(jnp.int32, (n, n), 0)
    c = lax.broadcasted_iota(jnp.int32, (n, n), 1)
    return (r >= c).astype(jnp.bfloat16)


def _cumsum0(tri, mask):
    return jnp.dot(tri, mask.astype(jnp.bfloat16),
                   preferred_element_type=jnp.float32)


def _cumsum0_chunked(tri_ch, mask, ch):
    n, k = mask.shape
    off = jnp.zeros((1, k), jnp.float32)
    outs = []
    for c in range(n // ch):
        cs = _cumsum0(tri_ch, mask[c * ch:(c + 1) * ch]) + off
        outs.append(cs)
        off = cs[ch - 1:ch, :]
    return jnp.concatenate(outs, axis=0)


def _outer_body(x_ref, wgo_ref, wgi_ref, td_ref, sm_ref, losso_ref):
    x = x_ref[...]
    lane8 = lax.broadcasted_iota(jnp.int32, (N, EO), 1)

    logits = jnp.dot(x, wgo_ref[...], preferred_element_type=jnp.float32)
    mx = jnp.max(logits, axis=-1, keepdims=True)
    ex = jnp.exp(logits - mx)
    raw = ex / jnp.sum(ex, axis=-1, keepdims=True)

    g1, i1 = _first_argmax(raw, lane8, EO)
    mask1 = (lane8 == i1).astype(jnp.float32)
    raw_wo = raw * (1.0 - mask1)
    g2, i2 = _first_argmax(raw_wo, lane8, EO)
    mask2 = (lane8 == i2).astype(jnp.float32)
    denom = g1 + g2 + EPS
    g1n = g1 / denom
    g2n = g2 / denom

    loss_o = jnp.sum(jnp.mean(mask1, axis=0) * jnp.mean(raw, axis=0)) * 8.0

    tri256 = _tri(256)
    cs1 = _cumsum0_chunked(tri256, mask1, 256)
    pos1 = (cs1 - mask1) * mask1
    mask1c = mask1 * (pos1 < float(CAP_O)).astype(jnp.float32)
    count1 = jnp.sum(mask1c, axis=0, keepdims=True)
    m1flat = jnp.sum(mask1c, axis=-1, keepdims=True)
    p1f = jnp.sum(pos1, axis=-1, keepdims=True)
    g1f = g1n * m1flat

    cs2 = _cumsum0_chunked(tri256, mask2, 256)
    pos2 = ((cs2 - mask2) + count1) * mask2
    mask2c = mask2 * (pos2 < float(CAP_O)).astype(jnp.float32)
    m2flat = jnp.sum(mask2c, axis=-1, keepdims=True)
    p2f = jnp.sum(pos2, axis=-1, keepdims=True)
    g2f = g2n * m2flat

    wgi = wgi_ref[...]
    sm_parts = []
    for e in range(EO):
        li = jnp.dot(x, wgi[e], preferred_element_type=jnp.float32)
        mi = jnp.max(li, axis=-1, keepdims=True)
        ei_ = jnp.exp(li - mi)
        sm_parts.append(ei_ / jnp.sum(ei_, axis=-1, keepdims=True))
    sm_ref[...] = jnp.concatenate(sm_parts, axis=1)

    td_ref[...] = jnp.concatenate(
        [i1.astype(jnp.float32), p1f, m1flat, g1f,
         i2.astype(jnp.float32), p2f, m2flat, g2f], axis=1)
    losso_ref[...] = jnp.reshape(loss_o, (1, 1))


def _inner_body(td_ref, sm_ref, src_ref, tok1_ref, tok2_ref, ci_ref, cw_ref,
                lossi_ref):
    e = pl.program_id(0)
    td = td_ref[...]
    i1 = td[:, 0:1].astype(jnp.int32)
    p1 = td[:, 1:2].astype(jnp.int32)
    m1flat = td[:, 2:3]
    g1f = td[:, 3:4]
    i2 = td[:, 4:5].astype(jnp.int32)
    p2 = td[:, 5:6].astype(jnp.int32)
    m2flat = td[:, 6:7]
    g2f = td[:, 7:8]

    cap_io = lax.broadcasted_iota(jnp.int32, (N, CAP_O), 1)
    lane8i = lax.broadcasted_iota(jnp.int32, (CAP_O, EI), 1)
    rows_io = lax.broadcasted_iota(jnp.int32, (CAP_O, ROWS_E), 1)
    n_iota = lax.broadcasted_iota(jnp.int32, (1, N), 1).astype(jnp.float32)
    sel_r = lax.broadcasted_iota(jnp.int32, (EO * EI, EI), 0)
    sel_c = lax.broadcasted_iota(jnp.int32, (EO * EI, EI), 1)
    tri320 = _tri(CAP_O)
    smp = _split3(sm_ref[...])
    np3 = _split3(n_iota)
    g1fp = _split3(g1f)
    g2fp = _split3(g2f)

    sel1 = (i1 == e) & (m1flat > 0.0)
    oh1 = ((cap_io == p1) & sel1).astype(jnp.bfloat16)
    sel2 = (i2 == e) & (m2flat > 0.0)
    oh2 = ((cap_io == p2) & sel2).astype(jnp.bfloat16)
    oh = oh1 + oh2

    cnt = jnp.dot(jnp.ones((1, N), jnp.bfloat16), oh,
                  preferred_element_type=jnp.float32)
    src_o = (_dot_parts(oh, np3, (((1,), (0,)), ((), ())), False)
             + (1.0 - cnt) * float(SENT))

    sel_mat = (sel_r == sel_c + 8 * e).astype(jnp.float32)
    G_all = _dot_parts(oh, smp, (((0,), (0,)), ((), ())))
    G = jnp.dot(G_all, sel_mat, preferred_element_type=jnp.float32,
                precision=_PH)
    imp = (_dot_parts(oh1, g1fp, (((0,), (0,)), ((), ())))
           + _dot_parts(oh2, g2fp, (((0,), (0,)), ((), ()))))
    eq1 = (imp > 0.5).astype(jnp.float32)
    gt0 = (imp > 0.0).astype(jnp.float32)

    ig1, ii1 = _first_argmax(G, lane8i, EI)
    im1 = (lane8i == ii1).astype(jnp.float32) * eq1
    ig1 = ig1 * eq1
    G_wo = G * (1.0 - im1)
    ig2, ii2 = _first_argmax(G_wo, lane8i, EI)
    im2 = (lane8i == ii2).astype(jnp.float32) * gt0
    idenom = ig1 + ig2 + EPS
    ig1n = ig1 / idenom
    ig2n = ig2 / idenom

    loss_e = jnp.sum(jnp.mean(im1, axis=0) * jnp.mean(G * eq1, axis=0))

    csi1 = _cumsum0(tri320, im1)
    ipos1 = (csi1 - im1) * im1
    im1c = im1 * (ipos1 < float(CAP_I)).astype(jnp.float32)
    icount1 = jnp.sum(im1c, axis=0, keepdims=True)
    im1flat = jnp.sum(im1c, axis=-1, keepdims=True)
    q1 = jnp.sum(ipos1, axis=-1, keepdims=True).astype(jnp.int32)
    h1 = ig1n * im1flat

    csi2 = _cumsum0(tri320, im2)
    ipos2 = ((csi2 - im2) + icount1) * im2
    im2c = im2 * (ipos2 < float(CAP_I)).astype(jnp.float32)
    im2flat = jnp.sum(im2c, axis=-1, keepdims=True)
    q2 = jnp.sum(ipos2, axis=-1, keepdims=True).astype(jnp.int32)
    h2 = ig2n * im2flat

    r1 = ii1 * CAP_I_PAD + q1
    r2 = ii2 * CAP_I_PAD + q2
    ohi = (((rows_io == r1) & (im1flat > 0.0)).astype(jnp.bfloat16)
           + ((rows_io == r2) & (im2flat > 0.0)).astype(jnp.bfloat16))
    cnti = jnp.dot(jnp.ones((1, CAP_O), jnp.bfloat16), ohi,
                   preferred_element_type=jnp.float32)
    src_e = (_dot_split(src_o, ohi, (((1,), (0,)), ((), ())), False)
             + (1.0 - cnti) * float(SENT))
    src_ref[...] = jnp.reshape(src_e.astype(jnp.int32), (1, 1, ROWS_E))

    base = (e * ROWS_E).astype(jnp.float32)
    V = jnp.concatenate(
        [h1, h2, r1.astype(jnp.float32) + base,
         r2.astype(jnp.float32) + base], axis=1)
    d1 = _dot_split(oh1, V, (((1,), (0,)), ((), ())), True)
    d2 = _dot_split(oh2, V, (((1,), (0,)), ((), ())), True)

    @pl.when(e == 0)
    def _():
        tok1_ref[...] = jnp.zeros((N, 4), jnp.float32)
        tok2_ref[...] = jnp.zeros((N, 4), jnp.float32)
        lossi_ref[...] = jnp.zeros((1, 1), jnp.float32)

    tok1_ref[...] += d1
    tok2_ref[...] += d2
    lossi_ref[...] += jnp.reshape(loss_e, (1, 1))

    @pl.when(e == EO - 1)
    def _():
        tok1 = tok1_ref[...]
        tok2 = tok2_ref[...]
        cw = jnp.concatenate(
            [g1f * tok1[:, 0:1], g1f * tok1[:, 1:2],
             g2f * tok2[:, 0:1], g2f * tok2[:, 1:2]], axis=1)
        ci = jnp.concatenate(
            [tok1[:, 2:3], tok1[:, 3:4], tok2[:, 2:3], tok2[:, 3:4]],
            axis=1).astype(jnp.int32)
        ci_ref[...] = jnp.clip(jnp.where(cw != 0.0, ci, 0), 0, ROWS - 1)
        cw_ref[...] = cw


def _ffn_body(x_ref, w1_ref, w2_ref, y_ref):
    h = jnp.maximum(
        jnp.dot(x_ref[...], w1_ref[0], preferred_element_type=jnp.float32),
        0.0)
    y_ref[...] = jnp.dot(h, w2_ref[0],
                         preferred_element_type=jnp.float32
                         ).astype(jnp.bfloat16)


def _combine_body(ci_ref, cw_ref, y_ref, out_ref):
    ci = ci_ref[...]
    cw = cw_ref[...]
    iot = lax.broadcasted_iota(jnp.int32, (N // 8, ROWS), 1)
    w = jnp.where(iot == ci[:, 0:1], cw[:, 0:1], 0.0)
    for j in range(1, 4):
        w = w + jnp.where(iot == ci[:, j:j + 1], cw[:, j:j + 1], 0.0)
    out_ref[...] = jnp.dot(w.astype(jnp.bfloat16), y_ref[...],
                           preferred_element_type=jnp.float32)


def _sc_gather(table, idx):
    b = idx.shape[0]
    d = table.shape[1]
    nw = 32
    b_per_w = b // nw
    ch = 64
    nch = b_per_w // ch
    idx2d = idx.reshape(b // ch, ch)
    mesh = plsc.VectorSubcoreMesh(core_axis_name="c", subcore_axis_name="s")

    @functools.partial(
        pl.kernel, mesh=mesh,
        out_type=jax.ShapeDtypeStruct((b, d), jnp.float32),
        scratch_types=[
            pltpu.VMEM((nch, ch), jnp.int32),
            pltpu.VMEM((ch, d), jnp.float32),
            pltpu.VMEM((ch, d), jnp.float32),
            pltpu.SemaphoreType.DMA,
            pltpu.SemaphoreType.DMA,
            pltpu.SemaphoreType.DMA,
            pltpu.SemaphoreType.DMA,
        ])
    def k(table_hbm, idx_hbm, out_hbm, idx_v, rows0, rows1, g0, g1, s0, s1):
        wid = lax.axis_index("s") * 2 + lax.axis_index("c")
        rows = (rows0, rows1)
        gsem = (g0, g1)
        ssem = (s0, s1)
        pltpu.sync_copy(idx_hbm.at[pl.ds(wid * nch, nch)], idx_v)

        def gather(c):
            return pltpu.async_copy(
                table_hbm.at[idx_v.at[c]], rows[c % 2], gsem[c % 2])

        def store(c):
            return pltpu.async_copy(
                rows[c % 2],
                out_hbm.at[pl.ds(wid * b_per_w + c * ch, ch)],
                ssem[c % 2])

        gathers = [None] * nch
        stores = [None] * nch
        gathers[0] = gather(0)
        for c in range(nch):
            gathers[c].wait()
            if c + 1 < nch:
                if c >= 1:
                    stores[c - 1].wait()
                gathers[c + 1] = gather(c + 1)
            stores[c] = store(c)
        for c in range(max(0, nch - 2), nch):
            stores[c].wait()

    return k(table, idx2d)


_outer_call = pl.pallas_call(
    _outer_body,
    out_shape=(
        jax.ShapeDtypeStruct((N, 8), jnp.float32),
        jax.ShapeDtypeStruct((N, EO * EI), jnp.float32),
        jax.ShapeDtypeStruct((1, 1), jnp.float32),
    ))

_inner_call = pl.pallas_call(
    _inner_body,
    grid=(EO,),
    in_specs=[
        pl.BlockSpec((N, 8), lambda e: (0, 0)),
        pl.BlockSpec((N, EO * EI), lambda e: (0, 0)),
    ],
    out_specs=(
        pl.BlockSpec((1, 1, ROWS_E), lambda e: (e, 0, 0)),
        pl.BlockSpec((N, 4), lambda e: (0, 0)),
        pl.BlockSpec((N, 4), lambda e: (0, 0)),
        pl.BlockSpec((N, 4), lambda e: (0, 0)),
        pl.BlockSpec((N, 4), lambda e: (0, 0)),
        pl.BlockSpec((1, 1), lambda e: (0, 0)),
    ),
    out_shape=(
        jax.ShapeDtypeStruct((EO, 1, ROWS_E), jnp.int32),
        jax.ShapeDtypeStruct((N, 4), jnp.float32),
        jax.ShapeDtypeStruct((N, 4), jnp.float32),
        jax.ShapeDtypeStruct((N, 4), jnp.int32),
        jax.ShapeDtypeStruct((N, 4), jnp.float32),
        jax.ShapeDtypeStruct((1, 1), jnp.float32),
    ))

_ffn_call = pl.pallas_call(
    _ffn_body,
    grid=(EO * EI,),
    in_specs=[
        pl.BlockSpec((CAP_I_PAD, DIM), lambda i: (i, 0)),
        pl.BlockSpec((1, DIM, HIDDEN), lambda i: (i, 0, 0)),
        pl.BlockSpec((1, HIDDEN, DIM), lambda i: (i, 0, 0)),
    ],
    out_specs=pl.BlockSpec((CAP_I_PAD, DIM), lambda i: (i, 0)),
    out_shape=jax.ShapeDtypeStruct((ROWS, DIM), jnp.bfloat16))

_combine_call = pl.pallas_call(
    _combine_body,
    grid=(8,),
    in_specs=[
        pl.BlockSpec((N // 8, 4), lambda i: (i, 0)),
        pl.BlockSpec((N // 8, 4), lambda i: (i, 0)),
        pl.BlockSpec((ROWS, DIM), lambda i: (0, 0)),
    ],
    out_specs=pl.BlockSpec((N // 8, DIM), lambda i: (i, 0)),
    out_shape=jax.ShapeDtypeStruct((N, DIM), jnp.float32))


def kernel(inputs, wg_outer, wg_inner, w1, w2):
    x2d = inputs.reshape(N, DIM)
    td, sm_all, loss_o = _outer_call(x2d, wg_outer, wg_inner)
    src, _t1, _t2, ci, cw, loss_i = _inner_call(td, sm_all)
    loss = (loss_o + loss_i) * 0.01

    xpad = jnp.concatenate([x2d, jnp.zeros((8, DIM), jnp.float32)], axis=0)
    xg = _sc_gather(xpad, src.reshape(ROWS))

    y = _ffn_call(xg, w1.reshape(EO * EI, DIM, HIDDEN),
                  w2.reshape(EO * EI, HIDDEN, DIM))

    out = _combine_call(ci, cw, y)
    return out.reshape(1, N, DIM), loss[0, 0]
